# baseline (speedup 1.0000x reference)
import functools

import numpy as np
import jax
import jax.numpy as jnp
from jax import lax
from jax.experimental import pallas as pl
from jax.experimental.pallas import tpu as pltpu
from jax.experimental.pallas import tpu_sc as plsc

_B, _N, _K, _DIM, _OUT = 16384, 1000000, 20, 32, 64
_L2 = 2 * _DIM
_M = _DIM + 2
_KD = _K * _DIM
_R = 256

_NW = 32
_BK = _B * _K
_PER_W = _BK // _NW
_CHUNK = 1024
_NCH = _PER_W // _CHUNK
_IPD = 128
_IROWS = _PER_W // _IPD


def _band_t5():
    t = np.arange(5)[:, None, None]
    d = np.arange(_DIM)[None, :, None]
    m = np.arange(_M)[None, None, :]
    return (d == m + t - 4).astype(np.float32)


def _band_u5():
    t = np.arange(5)[:, None, None]
    d = np.arange(_DIM)[None, :, None]
    l = np.arange(_L2)[None, None, :]
    return (d == l + t - 2).astype(np.float32)


def _band_o3():
    t = np.arange(3)[:, None, None]
    d = np.arange(_DIM)[None, :, None]
    l = np.arange(_DIM)[None, None, :]
    return (d == l + t - 1).astype(np.float32)


def _ttile():
    m = np.zeros((_DIM, _KD), np.float32)
    for k in range(_K):
        m[np.arange(_DIM), k * _DIM + np.arange(_DIM)] = 1.0
    return m


def _s640():
    m = np.zeros((_KD, _K), np.float32)
    for k in range(_K):
        m[k * _DIM + np.arange(_DIM), k] = 1.0
    return m


def _o20():
    m = np.zeros((_K, _KD), np.float32)
    for k in range(_K):
        m[k, k * _DIM:(k + 1) * _DIM] = 1.0
    return m


def _exp40():
    m = np.zeros((2 * _K, 32 * _DIM), np.float32)
    for o in range(_K):
        m[o * 2, o * _DIM + 30] = 1.0
        m[o * 2 + 1, o * _DIM + 31] = 1.0
    return m


_T5 = _band_t5()
_U5 = _band_u5()
_O3 = _band_o3()
_TTILE = _ttile()
_S640 = _s640()
_O20 = _o20()
_EXP40 = _exp40()


_PC = 8192
_PS = _PC // 4
_PG = (_N + _PC - 1) // _PC
_NPAD = _PG * _PC


def _e512():
    m = np.zeros((_DIM, 4 * 128), np.float32)
    for a in range(4):
        m[np.arange(_DIM), 128 * a + 32 * a + np.arange(_DIM)] = 1.0
    return m


def _prep_body(xt_ref, mt_ref, eye_ref, out_ref):
    prod = xt_ref[...] * mt_ref[...]
    col0 = pl.program_id(0) * _PC
    cid = lax.broadcasted_iota(jnp.int32, (_DIM, _PC), 1)
    prod = jnp.where(cid < _N - col0, prod, 0.0)
    eye = eye_ref[...]
    acc = None
    for a in range(4):
        pa = lax.dot_general(prod[:, a * _PS:(a + 1) * _PS],
                             eye[:, a * 128:(a + 1) * 128],
                             dimension_numbers=(((0,), (0,)), ((), ())),
                             preferred_element_type=jnp.float32)
        acc = pa if a == 0 else acc + pa
    out_ref[...] = acc


def _prep_call(xt, mt):
    return pl.pallas_call(
        _prep_body,
        grid=(_PG,),
        in_specs=[
            pl.BlockSpec((_DIM, _PC), lambda i: (0, i)),
            pl.BlockSpec((_DIM, _PC), lambda i: (0, i)),
            pl.BlockSpec((_DIM, 4 * 128), lambda i: (0, 0)),
        ],
        out_specs=pl.BlockSpec((_PS, 4 * _DIM), lambda i: (i, 0)),
        out_shape=jax.ShapeDtypeStruct((_PG * _PS, 4 * _DIM), jnp.float32),
    )(xt, mt, jnp.asarray(_e512()))


def _sc_gather(pt, y_train, idx2d, idy2d):
    info = plsc.get_sparse_core_info()
    nc = info.num_cores
    mesh = plsc.VectorSubcoreMesh(core_axis_name="c", subcore_axis_name="s")

    @functools.partial(
        pl.kernel,
        mesh=mesh,
        out_type=[
            jax.ShapeDtypeStruct((_BK, _DIM), jnp.float32),
            jax.ShapeDtypeStruct((_BK // _IPD, _IPD), jnp.float32),
        ],
        scratch_types=[
            pltpu.VMEM((_IROWS, _IPD), jnp.int32),
            pltpu.VMEM((_IROWS, _IPD), jnp.int32),
            pltpu.VMEM((_CHUNK, _DIM), jnp.float32),
            pltpu.VMEM((_CHUNK // _IPD, _IPD), jnp.float32),
            pltpu.SemaphoreType.DMA,
        ],
        compiler_params=pltpu.CompilerParams(use_tc_tiling_on_sc=False),
    )
    def k(pt_hbm, yt_hbm, idx_hbm, idy_hbm, cand_hbm, yg_hbm,
          idx_v, idy_v, cand_v, y_v, sem):
        wid = lax.axis_index("s") * nc + lax.axis_index("c")
        irow0 = wid * _IROWS
        row0 = wid * _PER_W
        pltpu.sync_copy(idx_hbm.at[pl.ds(irow0, _IROWS)], idx_v)
        pltpu.sync_copy(idy_hbm.at[pl.ds(irow0, _IROWS)], idy_v)
        dpc = _CHUNK // _IPD
        for c in range(_NCH):
            handles = []
            for j in range(dpc):
                ir = c * dpc + j
                idx_row = idx_v.at[ir]
                h1 = pltpu.make_async_copy(
                    pt_hbm.at[idx_row], cand_v.at[pl.ds(j * _IPD, _IPD)], sem)
                h3 = pltpu.make_async_copy(yt_hbm.at[idy_v.at[ir]], y_v.at[j], sem)
                h1.start(); h3.start()
                handles += [h1, h3]
            for h in handles:
                h.wait()
            r0 = row0 + c * _CHUNK
            pltpu.sync_copy(cand_v, cand_hbm.at[pl.ds(r0, _CHUNK)])
            pltpu.sync_copy(y_v, yg_hbm.at[pl.ds(irow0 + c * dpc, dpc)])

    return k(pt, y_train, idx2d, idy2d)


_G = 32
_GW = _G * _DIM


def _tree_max(v):
    s = _GW // 2
    while s >= _DIM:
        v = jnp.maximum(v[:, :s], v[:, s:])
        s //= 2
    return v


def _trans(v, eye):
    return lax.dot_general(v, eye, dimension_numbers=(((0,), (0,)), ((), ())),
                           preferred_element_type=jnp.float32)


def _tc_body(xt_ref, dmbt_ref, cand_ref, yg_ref, ndt_ref,
             wq_ref, s_ref, o20_ref, gn32_ref, gne_ref, exp40_ref,
             gxlo_ref, gxhi_ref, bias_ref,
             g1_ref, c1b_ref, w1p_ref, w1x_ref, w1ia_ref, w1ib_ref, w1n_ref,
             b1_ref, w2_ref, b2_ref, w3_ref, b3_ref, e32_ref, e20_ref, er_ref,
             out_ref, a_ref, e_ref):
    f32 = jnp.float32
    e32 = e32_ref[...]
    x = _trans(xt_ref[...], e32)
    P = cand_ref[...]
    xm = x * _trans(dmbt_ref[...], e32)
    qt = jnp.dot(xm, wq_ref[...], preferred_element_type=f32)
    e = jnp.dot(P * qt, s_ref[...], preferred_element_type=f32) * 0.125
    em = jnp.max(e, axis=1, keepdims=True)
    ea = jnp.exp(e - em)
    er = er_ref[...]
    a_ref[...] = _trans(ea / jnp.sum(ea, axis=1, keepdims=True), er)
    e_ref[...] = _trans(e, er)
    nd = _trans(ndt_ref[...], e20_ref[...])
    w = 1.0 / (nd + 1e-8)
    wn = w / jnp.sum(w, axis=1, keepdims=True)
    pred = jnp.sum(yg_ref[...] * wn, axis=1, keepdims=True)
    wt = jnp.dot(wn, o20_ref[...], preferred_element_type=f32)
    no = (P * wt).astype(jnp.bfloat16)
    n32 = jnp.dot(no, gn32_ref[...], preferred_element_type=f32)
    n5e = jnp.dot(no, gne_ref[...], preferred_element_type=f32)
    n5x = jnp.dot(n5e, exp40_ref[...], preferred_element_type=f32)
    cxlo = jnp.dot(x, gxlo_ref[...], preferred_element_type=f32)
    cxhi = jnp.dot(x, gxhi_ref[...], preferred_element_type=f32)
    x1c = jnp.dot(x, g1_ref[...], preferred_element_type=f32) + c1b_ref[...]
    bias = bias_ref[...]
    neigh = _tree_max(n32 + bias)
    im_a = _tree_max(cxlo + n5x + bias)
    im_b = _tree_max(cxhi + n32 + bias)
    x1 = (pred * w1p_ref[...]
          + jnp.dot(x1c, w1x_ref[...], preferred_element_type=f32)
          + jnp.dot(im_a, w1ia_ref[...], preferred_element_type=f32)
          + jnp.dot(im_b, w1ib_ref[...], preferred_element_type=f32)
          + jnp.dot(neigh, w1n_ref[...], preferred_element_type=f32)
          + b1_ref[...])
    x2 = jnp.maximum(jnp.dot(x1, w2_ref[...], preferred_element_type=f32)
                     + b2_ref[...], 0.0)
    x3 = jnp.maximum(jnp.dot(x2, w3_ref[...], preferred_element_type=f32)
                     + b3_ref[...], 0.0)
    out_ref[...] = pred + x3


def _tc_call(xt, dmbt, cand, yg, ndt, wq, s, o20, gn32, gne, exp40,
             gxlo, gxhi, bias,
             g1, c1b, w1p, w1x, w1ia, w1ib, w1n, b1, w2, b2, w3, b3):
    grid = (_B // _R,)

    def rows(c):
        return pl.BlockSpec((_R, c), lambda i: (i, 0))

    def cols(r):
        return pl.BlockSpec((r, _R), lambda i: (0, i))

    def full(a):
        return pl.BlockSpec(a.shape, lambda i: (0,) * a.ndim)

    e32 = jnp.eye(_DIM, dtype=jnp.float32)
    e20 = jnp.eye(_K, dtype=jnp.float32)
    er = jnp.eye(_R, dtype=jnp.float32)
    return pl.pallas_call(
        _tc_body,
        grid=grid,
        in_specs=[
            cols(_DIM), cols(_DIM), rows(_KD), rows(_K), cols(_K),
            full(wq), full(s), full(o20), full(gn32), full(gne), full(exp40),
            full(gxlo), full(gxhi), full(bias),
            full(g1), full(c1b), full(w1p), full(w1x),
            full(w1ia), full(w1ib), full(w1n), full(b1), full(w2),
            full(b2), full(w3), full(b3), full(e32), full(e20), full(er),
        ],
        out_specs=[
            pl.BlockSpec((_R, 1), lambda i: (i, 0)),
            pl.BlockSpec((_K, _R), lambda i: (0, i)),
            pl.BlockSpec((_K, _R), lambda i: (0, i)),
        ],
        out_shape=[
            jax.ShapeDtypeStruct((_B, 1), jnp.float32),
            jax.ShapeDtypeStruct((_K, _B), jnp.float32),
            jax.ShapeDtypeStruct((_K, _B), jnp.float32),
        ],
    )(xt, dmbt, cand, yg, ndt, wq, s, o20, gn32, gne, exp40, gxlo, gxhi, bias,
      g1, c1b, w1p, w1x, w1ia, w1ib, w1n, b1, w2, b2, w3, b3, e32, e20, er)


def kernel(X, X_train, neigh_ind, y_train, neigh_dist, data_m_train, data_m_batch,
           Wq, Wk, conv1_w, conv1_b, convn_w, convn_b, W1, b1, W2, b2, W3, b3):
    idx = neigh_ind.astype(jnp.int32)
    c = jnp.bitwise_and(idx, _PC - 1)
    rho = (idx - c) + 4 * jnp.bitwise_and(c, _PS - 1) + (c >> 11)
    idx2d = rho.reshape(_BK // _IPD, _IPD)
    idy2d = idx.reshape(_BK // _IPD, _IPD)
    pt = _prep_call(X_train.T, data_m_train.T).reshape(_NPAD, _DIM)
    cand, yg = _sc_gather(pt, y_train, idx2d, idy2d)
    cand = cand.reshape(_B, _KD)
    yg = yg.reshape(_B, _K)

    wq640 = (Wq @ Wk.T) @ _TTILE
    pad = ((0, 0), (0, (_G - _K) * _DIM))
    gn32 = jnp.einsum('okt,tdl->kdol', convn_w, _U5[:, :, :_DIM]).reshape(_KD, _KD)
    gn32 = jnp.pad(gn32, pad).astype(jnp.bfloat16)
    gne = jnp.einsum('okt,tdm->kdom', convn_w, _T5[:, :, :2]).reshape(_KD, 2 * _K)
    gne = gne.astype(jnp.bfloat16)
    a5 = convn_w.sum(axis=1)
    gxlo = jnp.pad(jnp.einsum('ot,tdl->dol', a5, _U5[:, :, :_DIM]).reshape(_DIM, _KD), pad)
    gxhi = jnp.pad(jnp.einsum('ot,tdl->dol', a5, _U5[:, :, _DIM:]).reshape(_DIM, _KD), pad)
    bias = jnp.concatenate([jnp.repeat(convn_b, _DIM),
                            jnp.full(((_G - _K) * _DIM,), -1e30, jnp.float32)]).reshape(1, _GW)
    g1 = jnp.einsum('t,tdl->dl', conv1_w[0, 0, :], _O3)
    c1b = conv1_b.reshape(1, 1)
    w1t = W1.T
    w1p = w1t[0:1]
    w1x = w1t[1:1 + _DIM]
    w1ia = w1t[33:65]
    w1ib = w1t[65:97]
    w1n = w1t[97:129]
    b1r = b1.reshape(1, -1)
    w2t = W2.T
    b2r = b2.reshape(1, -1)
    w3t = W3.T
    b3r = b3.reshape(1, -1)

    out1, at, et = _tc_call(X.T, data_m_batch.T, cand, yg, neigh_dist.T,
                          wq640, jnp.asarray(_S640), jnp.asarray(_O20),
                          gn32, gne, jnp.asarray(_EXP40), gxlo, gxhi, bias,
                          g1, c1b, w1p, w1x, w1ia, w1ib,
                          w1n, b1r, w2t, b2r, w3t, b3r)
    return (out1.reshape(-1), at.T, et.T)


# speedup vs baseline: 1.1068x; 1.1068x over previous
import functools

import numpy as np
import jax
import jax.numpy as jnp
from jax import lax
from jax.experimental import pallas as pl
from jax.experimental.pallas import tpu as pltpu
from jax.experimental.pallas import tpu_sc as plsc

_B, _N, _K, _DIM, _OUT = 16384, 1000000, 20, 32, 64
_L2 = 2 * _DIM
_M = _DIM + 2
_KD = _K * _DIM
_R = 512

_NW = 32
_BK = _B * _K
_PER_W = _BK // _NW
_CHUNK = 1024
_NCH = _PER_W // _CHUNK
_IPD = 128
_IROWS = _PER_W // _IPD


def _band_t5():
    t = np.arange(5)[:, None, None]
    d = np.arange(_DIM)[None, :, None]
    m = np.arange(_M)[None, None, :]
    return (d == m + t - 4).astype(np.float32)


def _band_u5():
    t = np.arange(5)[:, None, None]
    d = np.arange(_DIM)[None, :, None]
    l = np.arange(_L2)[None, None, :]
    return (d == l + t - 2).astype(np.float32)


def _band_o3():
    t = np.arange(3)[:, None, None]
    d = np.arange(_DIM)[None, :, None]
    l = np.arange(_DIM)[None, None, :]
    return (d == l + t - 1).astype(np.float32)


def _ttile():
    m = np.zeros((_DIM, _KD), np.float32)
    for k in range(_K):
        m[np.arange(_DIM), k * _DIM + np.arange(_DIM)] = 1.0
    return m


def _s640():
    m = np.zeros((_KD, _K), np.float32)
    for k in range(_K):
        m[k * _DIM + np.arange(_DIM), k] = 1.0
    return m


def _o20():
    m = np.zeros((_K, _KD), np.float32)
    for k in range(_K):
        m[k, k * _DIM:(k + 1) * _DIM] = 1.0
    return m


def _exp40():
    m = np.zeros((2 * _K, 32 * _DIM), np.float32)
    for o in range(_K):
        m[o * 2, o * _DIM + 30] = 1.0
        m[o * 2 + 1, o * _DIM + 31] = 1.0
    return m


_T5 = _band_t5()
_U5 = _band_u5()
_O3 = _band_o3()
_TTILE = _ttile()
_S640 = _s640()
_O20 = _o20()
_EXP40 = _exp40()


_PC = 8192
_PS = _PC // 4
_PG = (_N + _PC - 1) // _PC
_NPAD = _PG * _PC


def _e512():
    m = np.zeros((_DIM, 4 * 128), np.float32)
    for a in range(4):
        m[np.arange(_DIM), 128 * a + 32 * a + np.arange(_DIM)] = 1.0
    return m


def _prep_body(xt_ref, mt_ref, eye_ref, out_ref):
    prod = xt_ref[...] * mt_ref[...]
    col0 = pl.program_id(0) * _PC
    cid = lax.broadcasted_iota(jnp.int32, (_DIM, _PC), 1)
    prod = jnp.where(cid < _N - col0, prod, 0.0)
    eye = eye_ref[...]
    acc = None
    for a in range(4):
        pa = lax.dot_general(prod[:, a * _PS:(a + 1) * _PS],
                             eye[:, a * 128:(a + 1) * 128],
                             dimension_numbers=(((0,), (0,)), ((), ())),
                             preferred_element_type=jnp.float32)
        acc = pa if a == 0 else acc + pa
    out_ref[...] = acc


def _prep_call(xt, mt):
    return pl.pallas_call(
        _prep_body,
        grid=(_PG,),
        in_specs=[
            pl.BlockSpec((_DIM, _PC), lambda i: (0, i)),
            pl.BlockSpec((_DIM, _PC), lambda i: (0, i)),
            pl.BlockSpec((_DIM, 4 * 128), lambda i: (0, 0)),
        ],
        out_specs=pl.BlockSpec((_PS, 4 * _DIM), lambda i: (i, 0)),
        out_shape=jax.ShapeDtypeStruct((_PG * _PS, 4 * _DIM), jnp.float32),
    )(xt, mt, jnp.asarray(_e512()))


def _sc_gather(pt, y_train, idx2d, idy2d):
    info = plsc.get_sparse_core_info()
    nc = info.num_cores
    mesh = plsc.VectorSubcoreMesh(core_axis_name="c", subcore_axis_name="s")

    @functools.partial(
        pl.kernel,
        mesh=mesh,
        out_type=[
            jax.ShapeDtypeStruct((_BK, _DIM), jnp.float32),
            jax.ShapeDtypeStruct((_BK // _IPD, _IPD), jnp.float32),
        ],
        scratch_types=[
            pltpu.VMEM((_IROWS, _IPD), jnp.int32),
            pltpu.VMEM((_IROWS, _IPD), jnp.int32),
            pltpu.VMEM((_CHUNK, _DIM), jnp.float32),
            pltpu.VMEM((_CHUNK // _IPD, _IPD), jnp.float32),
            pltpu.SemaphoreType.DMA,
        ],
        compiler_params=pltpu.CompilerParams(use_tc_tiling_on_sc=False),
    )
    def k(pt_hbm, yt_hbm, idx_hbm, idy_hbm, cand_hbm, yg_hbm,
          idx_v, idy_v, cand_v, y_v, sem):
        wid = lax.axis_index("s") * nc + lax.axis_index("c")
        irow0 = wid * _IROWS
        row0 = wid * _PER_W
        pltpu.sync_copy(idx_hbm.at[pl.ds(irow0, _IROWS)], idx_v)
        pltpu.sync_copy(idy_hbm.at[pl.ds(irow0, _IROWS)], idy_v)
        dpc = _CHUNK // _IPD
        for c in range(_NCH):
            handles = []
            for j in range(dpc):
                ir = c * dpc + j
                idx_row = idx_v.at[ir]
                h1 = pltpu.make_async_copy(
                    pt_hbm.at[idx_row], cand_v.at[pl.ds(j * _IPD, _IPD)], sem)
                h3 = pltpu.make_async_copy(yt_hbm.at[idy_v.at[ir]], y_v.at[j], sem)
                h1.start(); h3.start()
                handles += [h1, h3]
            for h in handles:
                h.wait()
            r0 = row0 + c * _CHUNK
            pltpu.sync_copy(cand_v, cand_hbm.at[pl.ds(r0, _CHUNK)])
            pltpu.sync_copy(y_v, yg_hbm.at[pl.ds(irow0 + c * dpc, dpc)])

    return k(pt, y_train, idx2d, idy2d)


_G = 32
_GW = _G * _DIM


def _tree_max(v):
    s = _GW // 2
    while s >= _DIM:
        v = jnp.maximum(v[:, :s], v[:, s:])
        s //= 2
    return v


def _tc_body(x_ref, dmb_ref, cand_ref, yg_ref, nd_ref,
             wq_ref, s_ref, o20_ref, gn32_ref, gne_ref, exp40_ref,
             gxlo_ref, gxhi_ref, bias_ref,
             g1_ref, c1b_ref, w1p_ref, w1x_ref, w1ia_ref, w1ib_ref, w1n_ref,
             b1_ref, w2_ref, b2_ref, w3_ref, b3_ref,
             out_ref, a_ref, e_ref):
    f32 = jnp.float32
    x = x_ref[...]
    P = cand_ref[...]
    xm = x * dmb_ref[...]
    qt = jnp.dot(xm, wq_ref[...], preferred_element_type=f32)
    e = jnp.dot(P * qt, s_ref[...], preferred_element_type=f32) * 0.125
    em = jnp.max(e, axis=1, keepdims=True)
    ea = jnp.exp(e - em)
    a_ref[...] = ea / jnp.sum(ea, axis=1, keepdims=True)
    e_ref[...] = e
    w = 1.0 / (nd_ref[...] + 1e-8)
    wn = w / jnp.sum(w, axis=1, keepdims=True)
    pred = jnp.sum(yg_ref[...] * wn, axis=1, keepdims=True)
    wt = jnp.dot(wn, o20_ref[...], preferred_element_type=f32)
    no = (P * wt).astype(jnp.bfloat16)
    n32 = jnp.dot(no, gn32_ref[...], preferred_element_type=f32)
    n5e = jnp.dot(no, gne_ref[...], preferred_element_type=f32)
    n5x = jnp.dot(n5e, exp40_ref[...], preferred_element_type=f32)
    cxlo = jnp.dot(x, gxlo_ref[...], preferred_element_type=f32)
    cxhi = jnp.dot(x, gxhi_ref[...], preferred_element_type=f32)
    x1c = jnp.dot(x, g1_ref[...], preferred_element_type=f32) + c1b_ref[...]
    bias = bias_ref[...]
    neigh = _tree_max(n32 + bias)
    im_a = _tree_max(cxlo + n5x + bias)
    im_b = _tree_max(cxhi + n32 + bias)
    x1 = (pred * w1p_ref[...]
          + jnp.dot(x1c, w1x_ref[...], preferred_element_type=f32)
          + jnp.dot(im_a, w1ia_ref[...], preferred_element_type=f32)
          + jnp.dot(im_b, w1ib_ref[...], preferred_element_type=f32)
          + jnp.dot(neigh, w1n_ref[...], preferred_element_type=f32)
          + b1_ref[...])
    x2 = jnp.maximum(jnp.dot(x1, w2_ref[...], preferred_element_type=f32)
                     + b2_ref[...], 0.0)
    x3 = jnp.maximum(jnp.dot(x2, w3_ref[...], preferred_element_type=f32)
                     + b3_ref[...], 0.0)
    out_ref[...] = pred + x3


def _tc_call(x, dmb, cand, yg, nd, wq, s, o20, gn32, gne, exp40,
             gxlo, gxhi, bias,
             g1, c1b, w1p, w1x, w1ia, w1ib, w1n, b1, w2, b2, w3, b3):
    grid = (_B // _R,)

    def rows(c):
        return pl.BlockSpec((_R, c), lambda i: (i, 0))

    def full(a):
        return pl.BlockSpec(a.shape, lambda i: (0,) * a.ndim)

    return pl.pallas_call(
        _tc_body,
        grid=grid,
        in_specs=[
            rows(_DIM), rows(_DIM), rows(_KD), rows(_K), rows(_K),
            full(wq), full(s), full(o20), full(gn32), full(gne), full(exp40),
            full(gxlo), full(gxhi), full(bias),
            full(g1), full(c1b), full(w1p), full(w1x),
            full(w1ia), full(w1ib), full(w1n), full(b1), full(w2),
            full(b2), full(w3), full(b3),
        ],
        out_specs=[
            pl.BlockSpec((_R, 1), lambda i: (i, 0)),
            pl.BlockSpec((_R, _K), lambda i: (i, 0)),
            pl.BlockSpec((_R, _K), lambda i: (i, 0)),
        ],
        out_shape=[
            jax.ShapeDtypeStruct((_B, 1), jnp.float32),
            jax.ShapeDtypeStruct((_B, _K), jnp.float32),
            jax.ShapeDtypeStruct((_B, _K), jnp.float32),
        ],
    )(x, dmb, cand, yg, nd, wq, s, o20, gn32, gne, exp40, gxlo, gxhi, bias,
      g1, c1b, w1p, w1x, w1ia, w1ib, w1n, b1, w2, b2, w3, b3)


def kernel(X, X_train, neigh_ind, y_train, neigh_dist, data_m_train, data_m_batch,
           Wq, Wk, conv1_w, conv1_b, convn_w, convn_b, W1, b1, W2, b2, W3, b3):
    idx = neigh_ind.astype(jnp.int32)
    c = jnp.bitwise_and(idx, _PC - 1)
    rho = (idx - c) + 4 * jnp.bitwise_and(c, _PS - 1) + (c >> 11)
    idx2d = rho.reshape(_BK // _IPD, _IPD)
    idy2d = idx.reshape(_BK // _IPD, _IPD)
    pt = _prep_call(X_train.T, data_m_train.T).reshape(_NPAD, _DIM)
    cand, yg = _sc_gather(pt, y_train, idx2d, idy2d)
    cand = cand.reshape(_B, _KD)
    yg = yg.reshape(_B, _K)

    wq640 = (Wq @ Wk.T) @ _TTILE
    pad = ((0, 0), (0, (_G - _K) * _DIM))
    gn32 = jnp.einsum('okt,tdl->kdol', convn_w, _U5[:, :, :_DIM]).reshape(_KD, _KD)
    gn32 = jnp.pad(gn32, pad).astype(jnp.bfloat16)
    gne = jnp.einsum('okt,tdm->kdom', convn_w, _T5[:, :, :2]).reshape(_KD, 2 * _K)
    gne = gne.astype(jnp.bfloat16)
    a5 = convn_w.sum(axis=1)
    gxlo = jnp.pad(jnp.einsum('ot,tdl->dol', a5, _U5[:, :, :_DIM]).reshape(_DIM, _KD), pad)
    gxhi = jnp.pad(jnp.einsum('ot,tdl->dol', a5, _U5[:, :, _DIM:]).reshape(_DIM, _KD), pad)
    bias = jnp.concatenate([jnp.repeat(convn_b, _DIM),
                            jnp.full(((_G - _K) * _DIM,), -1e30, jnp.float32)]).reshape(1, _GW)
    g1 = jnp.einsum('t,tdl->dl', conv1_w[0, 0, :], _O3)
    c1b = conv1_b.reshape(1, 1)
    w1t = W1.T
    w1p = w1t[0:1]
    w1x = w1t[1:1 + _DIM]
    w1ia = w1t[33:65]
    w1ib = w1t[65:97]
    w1n = w1t[97:129]
    b1r = b1.reshape(1, -1)
    w2t = W2.T
    b2r = b2.reshape(1, -1)
    w3t = W3.T
    b3r = b3.reshape(1, -1)

    out1, a, e = _tc_call(X, data_m_batch, cand, yg, neigh_dist,
                          wq640, jnp.asarray(_S640), jnp.asarray(_O20),
                          gn32, gne, jnp.asarray(_EXP40), gxlo, gxhi, bias,
                          g1, c1b, w1p, w1x, w1ia, w1ib,
                          w1n, b1r, w2t, b2r, w3t, b3r)
    return (out1.reshape(-1), a, e)


# speedup vs baseline: 1.1963x; 1.0808x over previous
import functools

import numpy as np
import jax
import jax.numpy as jnp
from jax import lax
from jax.experimental import pallas as pl
from jax.experimental.pallas import tpu as pltpu
from jax.experimental.pallas import tpu_sc as plsc

_B, _N, _K, _DIM, _OUT = 16384, 1000000, 20, 32, 64
_L2 = 2 * _DIM
_M = _DIM + 2
_KD = _K * _DIM
_R = 512

_NW = 32
_BK = _B * _K
_PER_W = _BK // _NW
_CHUNK = 1024
_NCH = _PER_W // _CHUNK
_IPD = 128
_IROWS = _PER_W // _IPD


def _band_t5():
    t = np.arange(5)[:, None, None]
    d = np.arange(_DIM)[None, :, None]
    m = np.arange(_M)[None, None, :]
    return (d == m + t - 4).astype(np.float32)


def _band_u5():
    t = np.arange(5)[:, None, None]
    d = np.arange(_DIM)[None, :, None]
    l = np.arange(_L2)[None, None, :]
    return (d == l + t - 2).astype(np.float32)


def _band_o3():
    t = np.arange(3)[:, None, None]
    d = np.arange(_DIM)[None, :, None]
    l = np.arange(_DIM)[None, None, :]
    return (d == l + t - 1).astype(np.float32)


def _ttile():
    m = np.zeros((_DIM, _KD), np.float32)
    for k in range(_K):
        m[np.arange(_DIM), k * _DIM + np.arange(_DIM)] = 1.0
    return m


def _s640():
    m = np.zeros((_KD, _K), np.float32)
    for k in range(_K):
        m[k * _DIM + np.arange(_DIM), k] = 1.0
    return m


def _o20():
    m = np.zeros((_K, _KD), np.float32)
    for k in range(_K):
        m[k, k * _DIM:(k + 1) * _DIM] = 1.0
    return m


def _exp40():
    m = np.zeros((2 * _K, 32 * _DIM), np.float32)
    for o in range(_K):
        m[o * 2, o * _DIM + 30] = 1.0
        m[o * 2 + 1, o * _DIM + 31] = 1.0
    return m


_T5 = _band_t5()
_U5 = _band_u5()
_O3 = _band_o3()
_TTILE = _ttile()
_S640 = _s640()
_O20 = _o20()
_EXP40 = _exp40()


_PC = 16384
_PS = _PC // 4
_PG = (_N + _PC - 1) // _PC
_NPAD = _PG * _PC
_PSH = _PS.bit_length() - 1


def _e512():
    m = np.zeros((_DIM, 4 * 128), np.float32)
    for a in range(4):
        m[np.arange(_DIM), 128 * a + 32 * a + np.arange(_DIM)] = 1.0
    return m


def _prep_body(xt_ref, mt_ref, eye_ref, out_ref):
    prod = xt_ref[...] * mt_ref[...]
    col0 = pl.program_id(0) * _PC
    cid = lax.broadcasted_iota(jnp.int32, (_DIM, _PC), 1)
    prod = jnp.where(cid < _N - col0, prod, 0.0)
    eye = eye_ref[...]
    acc = None
    for a in range(4):
        pa = lax.dot_general(prod[:, a * _PS:(a + 1) * _PS],
                             eye[:, a * 128:(a + 1) * 128],
                             dimension_numbers=(((0,), (0,)), ((), ())),
                             preferred_element_type=jnp.float32)
        acc = pa if a == 0 else acc + pa
    out_ref[...] = acc


def _prep_call(xt, mt):
    return pl.pallas_call(
        _prep_body,
        grid=(_PG,),
        in_specs=[
            pl.BlockSpec((_DIM, _PC), lambda i: (0, i)),
            pl.BlockSpec((_DIM, _PC), lambda i: (0, i)),
            pl.BlockSpec((_DIM, 4 * 128), lambda i: (0, 0)),
        ],
        out_specs=pl.BlockSpec((_PS, 4 * _DIM), lambda i: (i, 0)),
        out_shape=jax.ShapeDtypeStruct((_PG * _PS, 4 * _DIM), jnp.float32),
    )(xt, mt, jnp.asarray(_e512()))


def _sc_gather(pt, y_train, idx2d, idy2d):
    info = plsc.get_sparse_core_info()
    nc = info.num_cores
    mesh = plsc.VectorSubcoreMesh(core_axis_name="c", subcore_axis_name="s")

    @functools.partial(
        pl.kernel,
        mesh=mesh,
        out_type=[
            jax.ShapeDtypeStruct((_BK, _DIM), jnp.float32),
            jax.ShapeDtypeStruct((_BK // _IPD, _IPD), jnp.float32),
        ],
        scratch_types=[
            pltpu.VMEM((_IROWS, _IPD), jnp.int32),
            pltpu.VMEM((_IROWS, _IPD), jnp.int32),
            pltpu.VMEM((_CHUNK, _DIM), jnp.float32),
            pltpu.VMEM((_CHUNK // _IPD, _IPD), jnp.float32),
            pltpu.SemaphoreType.DMA,
        ],
        compiler_params=pltpu.CompilerParams(use_tc_tiling_on_sc=False),
    )
    def k(pt_hbm, yt_hbm, idx_hbm, idy_hbm, cand_hbm, yg_hbm,
          idx_v, idy_v, cand_v, y_v, sem):
        wid = lax.axis_index("s") * nc + lax.axis_index("c")
        irow0 = wid * _IROWS
        row0 = wid * _PER_W
        pltpu.sync_copy(idx_hbm.at[pl.ds(irow0, _IROWS)], idx_v)
        pltpu.sync_copy(idy_hbm.at[pl.ds(irow0, _IROWS)], idy_v)
        dpc = _CHUNK // _IPD
        for c in range(_NCH):
            handles = []
            for j in range(dpc):
                ir = c * dpc + j
                idx_row = idx_v.at[ir]
                h1 = pltpu.make_async_copy(
                    pt_hbm.at[idx_row], cand_v.at[pl.ds(j * _IPD, _IPD)], sem)
                h3 = pltpu.make_async_copy(yt_hbm.at[idy_v.at[ir]], y_v.at[j], sem)
                h1.start(); h3.start()
                handles += [h1, h3]
            for h in handles:
                h.wait()
            r0 = row0 + c * _CHUNK
            pltpu.sync_copy(cand_v, cand_hbm.at[pl.ds(r0, _CHUNK)])
            pltpu.sync_copy(y_v, yg_hbm.at[pl.ds(irow0 + c * dpc, dpc)])

    return k(pt, y_train, idx2d, idy2d)


_G = 32
_GW = _G * _DIM


def _tree_max(v):
    s = _GW // 2
    while s >= _DIM:
        v = jnp.maximum(v[:, :s], v[:, s:])
        s //= 2
    return v


def _tc_body(x_ref, dmb_ref, cand_ref, yg_ref, nd_ref,
             wq_ref, s_ref, o20_ref, gn32_ref, gne_ref, exp40_ref,
             gxlo_ref, gxhi_ref, bias_ref,
             g1_ref, c1b_ref, w1p_ref, w1x_ref, w1ia_ref, w1ib_ref, w1n_ref,
             b1_ref, w2_ref, b2_ref, w3_ref, b3_ref,
             out_ref, a_ref, e_ref):
    f32 = jnp.float32
    x = x_ref[...]
    P = cand_ref[...]
    xm = x * dmb_ref[...]
    qt = jnp.dot(xm, wq_ref[...], preferred_element_type=f32)
    e = jnp.dot(P * qt, s_ref[...], preferred_element_type=f32) * 0.125
    em = jnp.max(e, axis=1, keepdims=True)
    ea = jnp.exp(e - em)
    a_ref[...] = ea / jnp.sum(ea, axis=1, keepdims=True)
    e_ref[...] = e
    w = 1.0 / (nd_ref[...] + 1e-8)
    wn = w / jnp.sum(w, axis=1, keepdims=True)
    pred = jnp.sum(yg_ref[...] * wn, axis=1, keepdims=True)
    wt = jnp.dot(wn, o20_ref[...], preferred_element_type=f32)
    no = (P * wt).astype(jnp.bfloat16)
    n32 = jnp.dot(no, gn32_ref[...], preferred_element_type=f32)
    n5e = jnp.dot(no, gne_ref[...], preferred_element_type=f32)
    n5x = jnp.dot(n5e, exp40_ref[...], preferred_element_type=f32)
    cxlo = jnp.dot(x, gxlo_ref[...], preferred_element_type=f32)
    cxhi = jnp.dot(x, gxhi_ref[...], preferred_element_type=f32)
    x1c = jnp.dot(x, g1_ref[...], preferred_element_type=f32) + c1b_ref[...]
    bias = bias_ref[...]
    neigh = _tree_max(n32 + bias)
    im_a = _tree_max(cxlo + n5x + bias)
    im_b = _tree_max(cxhi + n32 + bias)
    x1 = (pred * w1p_ref[...]
          + jnp.dot(x1c, w1x_ref[...], preferred_element_type=f32)
          + jnp.dot(im_a, w1ia_ref[...], preferred_element_type=f32)
          + jnp.dot(im_b, w1ib_ref[...], preferred_element_type=f32)
          + jnp.dot(neigh, w1n_ref[...], preferred_element_type=f32)
          + b1_ref[...])
    x2 = jnp.maximum(jnp.dot(x1, w2_ref[...], preferred_element_type=f32)
                     + b2_ref[...], 0.0)
    x3 = jnp.maximum(jnp.dot(x2, w3_ref[...], preferred_element_type=f32)
                     + b3_ref[...], 0.0)
    out_ref[...] = pred + x3


def _tc_call(x, dmb, cand, yg, nd, wq, s, o20, gn32, gne, exp40,
             gxlo, gxhi, bias,
             g1, c1b, w1p, w1x, w1ia, w1ib, w1n, b1, w2, b2, w3, b3):
    grid = (_B // _R,)

    def rows(c):
        return pl.BlockSpec((_R, c), lambda i: (i, 0))

    def full(a):
        return pl.BlockSpec(a.shape, lambda i: (0,) * a.ndim)

    return pl.pallas_call(
        _tc_body,
        grid=grid,
        in_specs=[
            rows(_DIM), rows(_DIM), rows(_KD), rows(_K), rows(_K),
            full(wq), full(s), full(o20), full(gn32), full(gne), full(exp40),
            full(gxlo), full(gxhi), full(bias),
            full(g1), full(c1b), full(w1p), full(w1x),
            full(w1ia), full(w1ib), full(w1n), full(b1), full(w2),
            full(b2), full(w3), full(b3),
        ],
        out_specs=[
            pl.BlockSpec((_R, 1), lambda i: (i, 0)),
            pl.BlockSpec((_R, _K), lambda i: (i, 0)),
            pl.BlockSpec((_R, _K), lambda i: (i, 0)),
        ],
        out_shape=[
            jax.ShapeDtypeStruct((_B, 1), jnp.float32),
            jax.ShapeDtypeStruct((_B, _K), jnp.float32),
            jax.ShapeDtypeStruct((_B, _K), jnp.float32),
        ],
    )(x, dmb, cand, yg, nd, wq, s, o20, gn32, gne, exp40, gxlo, gxhi, bias,
      g1, c1b, w1p, w1x, w1ia, w1ib, w1n, b1, w2, b2, w3, b3)


def kernel(X, X_train, neigh_ind, y_train, neigh_dist, data_m_train, data_m_batch,
           Wq, Wk, conv1_w, conv1_b, convn_w, convn_b, W1, b1, W2, b2, W3, b3):
    idx = neigh_ind.astype(jnp.int32)
    c = jnp.bitwise_and(idx, _PC - 1)
    rho = (idx - c) + 4 * jnp.bitwise_and(c, _PS - 1) + (c >> _PSH)
    idx2d = rho.reshape(_BK // _IPD, _IPD)
    idy2d = idx.reshape(_BK // _IPD, _IPD)
    pt = _prep_call(X_train.T, data_m_train.T).reshape(_NPAD, _DIM)
    cand, yg = _sc_gather(pt, y_train, idx2d, idy2d)
    cand = cand.reshape(_B, _KD)
    yg = yg.reshape(_B, _K)

    wq640 = (Wq @ Wk.T) @ _TTILE
    pad = ((0, 0), (0, (_G - _K) * _DIM))
    gn32 = jnp.einsum('okt,tdl->kdol', convn_w, _U5[:, :, :_DIM]).reshape(_KD, _KD)
    gn32 = jnp.pad(gn32, pad).astype(jnp.bfloat16)
    gne = jnp.einsum('okt,tdm->kdom', convn_w, _T5[:, :, :2]).reshape(_KD, 2 * _K)
    gne = gne.astype(jnp.bfloat16)
    a5 = convn_w.sum(axis=1)
    gxlo = jnp.pad(jnp.einsum('ot,tdl->dol', a5, _U5[:, :, :_DIM]).reshape(_DIM, _KD), pad)
    gxhi = jnp.pad(jnp.einsum('ot,tdl->dol', a5, _U5[:, :, _DIM:]).reshape(_DIM, _KD), pad)
    bias = jnp.concatenate([jnp.repeat(convn_b, _DIM),
                            jnp.full(((_G - _K) * _DIM,), -1e30, jnp.float32)]).reshape(1, _GW)
    g1 = jnp.einsum('t,tdl->dl', conv1_w[0, 0, :], _O3)
    c1b = conv1_b.reshape(1, 1)
    w1t = W1.T
    w1p = w1t[0:1]
    w1x = w1t[1:1 + _DIM]
    w1ia = w1t[33:65]
    w1ib = w1t[65:97]
    w1n = w1t[97:129]
    b1r = b1.reshape(1, -1)
    w2t = W2.T
    b2r = b2.reshape(1, -1)
    w3t = W3.T
    b3r = b3.reshape(1, -1)

    out1, a, e = _tc_call(X, data_m_batch, cand, yg, neigh_dist,
                          wq640, jnp.asarray(_S640), jnp.asarray(_O20),
                          gn32, gne, jnp.asarray(_EXP40), gxlo, gxhi, bias,
                          g1, c1b, w1p, w1x, w1ia, w1ib,
                          w1n, b1r, w2t, b2r, w3t, b3r)
    return (out1.reshape(-1), a, e)
